# Initial kernel scaffold; baseline (speedup 1.0000x reference)
import jax
import jax.numpy as jnp
from jax.experimental import pallas as pl


def kernel(boxes, image_meta, feat_p2, feat_p3, feat_p4, feat_p5):
    raise NotImplementedError("write your pallas kernel here")


# speedup vs baseline: 14.6522x; 14.6522x over previous
import functools

import jax
import jax.numpy as jnp
from jax import lax
from jax.experimental import pallas as pl
from jax.experimental.pallas import tpu as pltpu
from jax.experimental.pallas import tpu_sc as plsc

POOL = 7
PP = POOL * POOL
GPAD = 224


def _prep(boxes, image_meta, feats):
    B, Nb, _ = boxes.shape
    C = feats[0].shape[-1]
    N = B * Nb

    hs = [f.shape[1] for f in feats]
    ws = [f.shape[2] for f in feats]
    sizes = [B * h * w for h, w in zip(hs, ws)]
    bases = [sum(sizes[:i]) for i in range(len(sizes))]

    table = jnp.concatenate([f.reshape(-1, C) for f in feats], axis=0)

    fb = boxes.reshape(-1, 4)
    bw = fb[:, 3] - fb[:, 1]
    bh = fb[:, 2] - fb[:, 0]
    image_shape = image_meta[0, 4:7]
    image_area = image_shape[0] * image_shape[1]
    eq = jnp.log(jnp.sqrt(jnp.maximum(bw * bh, 1e-12)) * jnp.sqrt(image_area) / 224.0) / jnp.log(2.0)
    levels = jnp.maximum(2, jnp.minimum(4 + jnp.round(eq).astype(jnp.int32), 5))
    li = levels - 2

    h_i = jnp.take(jnp.asarray(hs, jnp.int32), li)
    w_i = jnp.take(jnp.asarray(ws, jnp.int32), li)
    hw_i = jnp.take(jnp.asarray([h * w for h, w in zip(hs, ws)], jnp.int32), li)
    base = jnp.take(jnp.asarray(bases, jnp.int32), li)
    bb = jnp.repeat(jnp.arange(B, dtype=jnp.int32), Nb)
    base = base + bb * hw_i

    hf = h_i.astype(jnp.float32)
    wf = w_i.astype(jnp.float32)
    g = jnp.arange(POOL, dtype=jnp.float32)
    ys = fb[:, 0][:, None] * (hf - 1.0)[:, None] + g[None, :] * ((fb[:, 2] - fb[:, 0]) * (hf - 1.0) / (POOL - 1))[:, None]
    xs = fb[:, 1][:, None] * (wf - 1.0)[:, None] + g[None, :] * ((fb[:, 3] - fb[:, 1]) * (wf - 1.0) / (POOL - 1))[:, None]
    y0f = jnp.floor(ys)
    x0f = jnp.floor(xs)
    y0 = jnp.clip(y0f.astype(jnp.int32), 0, h_i[:, None] - 1)
    y1 = jnp.clip(y0 + 1, 0, h_i[:, None] - 1)
    x0 = jnp.clip(x0f.astype(jnp.int32), 0, w_i[:, None] - 1)
    x1 = jnp.clip(x0 + 1, 0, w_i[:, None] - 1)
    wy = ys - y0f
    wx = xs - x0f

    ry0 = base[:, None] + y0 * w_i[:, None]
    ry1 = base[:, None] + y1 * w_i[:, None]
    tl = ry0[:, :, None] + x0[:, None, :]
    tr = ry0[:, :, None] + x1[:, None, :]
    bl = ry1[:, :, None] + x0[:, None, :]
    br = ry1[:, :, None] + x1[:, None, :]
    inter = jnp.stack([tl, tr, bl, br], axis=-1).reshape(N, 4 * PP).astype(jnp.int32)
    pad = jnp.broadcast_to(inter[:, -1:], (N, GPAD - 4 * PP))
    idx = jnp.concatenate([inter, pad], axis=1)

    wx_plane = jnp.broadcast_to(wx[:, None, :, None], (N, POOL, POOL, 16)).reshape(N, PP * 16)
    wy_plane = jnp.broadcast_to(wy[:, :, None, None], (N, POOL, POOL, 16)).reshape(N, PP * 16)
    return table, idx, wx_plane, wy_plane


def kernel(boxes, image_meta, feat_p2, feat_p3, feat_p4, feat_p5):
    feats = [feat_p2, feat_p3, feat_p4, feat_p5]
    B, Nb, _ = boxes.shape
    C = feats[0].shape[-1]
    N = B * Nb
    assert C % 16 == 0

    table, idx, wx_plane, wy_plane = _prep(boxes, image_meta, feats)

    info = plsc.get_sparse_core_info()
    NC, NS = info.num_cores, info.num_subcores
    NW = NC * NS
    per_worker = -(-N // NW)

    mesh = plsc.VectorSubcoreMesh(core_axis_name="c", subcore_axis_name="s")

    @functools.partial(
        pl.kernel,
        mesh=mesh,
        out_type=jax.ShapeDtypeStruct((N, PP, C), jnp.float32),
        scratch_types=[
            pltpu.VMEM((GPAD,), jnp.int32),
            pltpu.VMEM((PP * 16,), jnp.float32),
            pltpu.VMEM((PP * 16,), jnp.float32),
            pltpu.VMEM((GPAD, C), jnp.float32),
            pltpu.VMEM((PP, C), jnp.float32),
            pltpu.SemaphoreType.DMA,
        ],
    )
    def sc_pool(table_h, idx_h, wx_h, wy_h, out_h,
                idx_v, wx_v, wy_v, rows_v, out_v, sem):
        wid = lax.axis_index("s") * NC + lax.axis_index("c")
        base_box = wid * per_worker

        def box_body(j, carry):
            box = base_box + j

            @pl.when(box < N)
            def _():
                pltpu.sync_copy(idx_h.at[box], idx_v)
                pltpu.sync_copy(wx_h.at[box], wx_v)
                pltpu.sync_copy(wy_h.at[box], wy_v)
                c0 = pltpu.async_copy(
                    table_h.at[idx_v.at[pl.ds(0, 112)]],
                    rows_v.at[pl.ds(0, 112)], sem)
                c1 = pltpu.async_copy(
                    table_h.at[idx_v.at[pl.ds(112, 112)]],
                    rows_v.at[pl.ds(112, 112)], sem)
                c0.wait()
                c1.wait()

                def pt_body(p, pcarry):
                    wxp = wx_v[pl.ds(p * 16, 16)]
                    wyp = wy_v[pl.ds(p * 16, 16)]
                    r = p * 4
                    for ch in range(C // 16):
                        s = pl.ds(ch * 16, 16)
                        tl = rows_v[r, s]
                        tr = rows_v[r + 1, s]
                        bl = rows_v[r + 2, s]
                        br = rows_v[r + 3, s]
                        top = tl + (tr - tl) * wxp
                        bot = bl + (br - bl) * wxp
                        out_v[p, s] = top + (bot - top) * wyp
                    return pcarry

                lax.fori_loop(0, PP, pt_body, 0)
                pltpu.sync_copy(out_v, out_h.at[box])

            return carry

        lax.fori_loop(0, per_worker, box_body, 0)

    out = sc_pool(table, idx, wx_plane, wy_plane)
    return out.reshape(B, Nb, POOL, POOL, C)


# speedup vs baseline: 15.9694x; 1.0899x over previous
import functools

import jax
import jax.numpy as jnp
from jax import lax
from jax.experimental import pallas as pl
from jax.experimental.pallas import tpu as pltpu
from jax.experimental.pallas import tpu_sc as plsc

POOL = 7
PP = POOL * POOL
GPAD = 224
WROW = 128


def _prep(boxes, image_meta, feats):
    B, Nb, _ = boxes.shape
    C = feats[0].shape[-1]
    N = B * Nb

    hs = [f.shape[1] for f in feats]
    ws = [f.shape[2] for f in feats]
    sizes = [B * h * w for h, w in zip(hs, ws)]
    bases = [sum(sizes[:i]) for i in range(len(sizes))]

    table = jnp.concatenate([f.reshape(-1, C) for f in feats], axis=0)

    fb = boxes.reshape(-1, 4)
    bw = fb[:, 3] - fb[:, 1]
    bh = fb[:, 2] - fb[:, 0]
    image_shape = image_meta[0, 4:7]
    image_area = image_shape[0] * image_shape[1]
    eq = jnp.log(jnp.sqrt(jnp.maximum(bw * bh, 1e-12)) * jnp.sqrt(image_area) / 224.0) / jnp.log(2.0)
    levels = jnp.maximum(2, jnp.minimum(4 + jnp.round(eq).astype(jnp.int32), 5))
    li = levels - 2

    h_i = jnp.take(jnp.asarray(hs, jnp.int32), li)
    w_i = jnp.take(jnp.asarray(ws, jnp.int32), li)
    hw_i = jnp.take(jnp.asarray([h * w for h, w in zip(hs, ws)], jnp.int32), li)
    base = jnp.take(jnp.asarray(bases, jnp.int32), li)
    bb = jnp.repeat(jnp.arange(B, dtype=jnp.int32), Nb)
    base = base + bb * hw_i

    hf = h_i.astype(jnp.float32)
    wf = w_i.astype(jnp.float32)
    g = jnp.arange(POOL, dtype=jnp.float32)
    ys = fb[:, 0][:, None] * (hf - 1.0)[:, None] + g[None, :] * ((fb[:, 2] - fb[:, 0]) * (hf - 1.0) / (POOL - 1))[:, None]
    xs = fb[:, 1][:, None] * (wf - 1.0)[:, None] + g[None, :] * ((fb[:, 3] - fb[:, 1]) * (wf - 1.0) / (POOL - 1))[:, None]
    y0f = jnp.floor(ys)
    x0f = jnp.floor(xs)
    y0 = jnp.clip(y0f.astype(jnp.int32), 0, h_i[:, None] - 1)
    y1 = jnp.clip(y0 + 1, 0, h_i[:, None] - 1)
    x0 = jnp.clip(x0f.astype(jnp.int32), 0, w_i[:, None] - 1)
    x1 = jnp.clip(x0 + 1, 0, w_i[:, None] - 1)
    wy = ys - y0f
    wx = xs - x0f

    ry0 = base[:, None] + y0 * w_i[:, None]
    ry1 = base[:, None] + y1 * w_i[:, None]
    tl = ry0[:, :, None] + x0[:, None, :]
    tr = ry0[:, :, None] + x1[:, None, :]
    bl = ry1[:, :, None] + x0[:, None, :]
    br = ry1[:, :, None] + x1[:, None, :]
    inter = jnp.stack([tl, tr, bl, br], axis=-1).reshape(N, 4 * PP).astype(jnp.int32)
    pad = jnp.broadcast_to(inter[:, -1:], (N, GPAD - 4 * PP))
    idx = jnp.concatenate([inter, pad], axis=1)

    wx_p = jnp.broadcast_to(wx[:, None, :], (N, POOL, POOL)).reshape(N, PP)
    wy_p = jnp.broadcast_to(wy[:, :, None], (N, POOL, POOL)).reshape(N, PP)
    zeros15 = jnp.zeros((N, 64 - PP), jnp.float32)
    wrow = jnp.concatenate([wx_p, zeros15, wy_p, zeros15], axis=1)
    return table, idx, wrow


def kernel(boxes, image_meta, feat_p2, feat_p3, feat_p4, feat_p5):
    feats = [feat_p2, feat_p3, feat_p4, feat_p5]
    B, Nb, _ = boxes.shape
    C = feats[0].shape[-1]
    N = B * Nb
    assert C % 16 == 0

    table, idx, wrow = _prep(boxes, image_meta, feats)

    info = plsc.get_sparse_core_info()
    NC, NS = info.num_cores, info.num_subcores
    NW = NC * NS
    per_worker = -(-N // NW)
    npad = NW * per_worker - N

    idx_t = jnp.concatenate([idx, jnp.zeros((npad, GPAD), jnp.int32)], axis=0)
    idx_t = idx_t.reshape(NW, per_worker * GPAD)
    wrow_t = jnp.concatenate([wrow, jnp.zeros((npad, WROW), jnp.float32)], axis=0)
    wrow_t = wrow_t.reshape(NW, per_worker * WROW)

    mesh = plsc.VectorSubcoreMesh(core_axis_name="c", subcore_axis_name="s")

    @functools.partial(
        pl.kernel,
        mesh=mesh,
        out_type=jax.ShapeDtypeStruct((N, PP, C), jnp.float32),
        scratch_types=[
            pltpu.VMEM((per_worker * GPAD,), jnp.int32),
            pltpu.VMEM((per_worker * WROW,), jnp.float32),
            pltpu.VMEM((GPAD, C), jnp.float32),
            pltpu.VMEM((PP, C), jnp.float32),
            pltpu.SemaphoreType.DMA,
        ],
    )
    def sc_pool(table_h, idx_h, w_h, out_h, idx_v, w_v, rows_v, out_v, sem):
        wid = lax.axis_index("s") * NC + lax.axis_index("c")
        base_box = wid * per_worker

        pltpu.sync_copy(idx_h.at[wid], idx_v)
        pltpu.sync_copy(w_h.at[wid], w_v)

        def box_body(j, carry):
            box = base_box + j

            @pl.when(box < N)
            def _():
                c0 = pltpu.async_copy(
                    table_h.at[idx_v.at[pl.ds(j * GPAD, 112)]],
                    rows_v.at[pl.ds(0, 112)], sem)
                c1 = pltpu.async_copy(
                    table_h.at[idx_v.at[pl.ds(j * GPAD + 112, 112)]],
                    rows_v.at[pl.ds(112, 112)], sem)
                c0.wait()
                c1.wait()

                wb = j * WROW

                dnums = lax.GatherDimensionNumbers(
                    offset_dims=(), collapsed_slice_dims=(0,), start_index_map=(0,))

                def bcast_lane(vec16, lane):
                    idxv = jnp.broadcast_to(lane, (16,)).astype(jnp.int32)
                    return lax.gather(vec16, idxv[:, None], dnums, (1,),
                                      mode=lax.GatherScatterMode.PROMISE_IN_BOUNDS)

                def pt_body(p, pcarry):
                    chunk = (p // 16) * 16
                    lane = p - chunk
                    wxc = w_v[pl.ds(wb + chunk, 16)]
                    wyc = w_v[pl.ds(wb + 64 + chunk, 16)]
                    wxp = bcast_lane(wxc, lane)
                    wyp = bcast_lane(wyc, lane)
                    r = p * 4
                    for ch in range(C // 16):
                        s = pl.ds(ch * 16, 16)
                        tl = rows_v[r, s]
                        tr = rows_v[r + 1, s]
                        bl = rows_v[r + 2, s]
                        br = rows_v[r + 3, s]
                        top = tl + (tr - tl) * wxp
                        bot = bl + (br - bl) * wxp
                        out_v[p, s] = top + (bot - top) * wyp
                    return pcarry

                lax.fori_loop(0, PP, pt_body, 0)
                pltpu.sync_copy(out_v, out_h.at[box])

            return carry

        lax.fori_loop(0, per_worker, box_body, 0)

    out = sc_pool(table, idx_t, wrow_t)
    return out.reshape(B, Nb, POOL, POOL, C)
